# Initial kernel scaffold; baseline (speedup 1.0000x reference)
import jax
import jax.numpy as jnp
from jax.experimental import pallas as pl


def kernel(x, embedding_weight):
    raise NotImplementedError("write your pallas kernel here")


# speedup vs baseline: 2.9861x; 2.9861x over previous
import functools

import jax
import jax.numpy as jnp
from jax import lax
from jax.experimental import pallas as pl
from jax.experimental.pallas import tpu as pltpu
from jax.experimental.pallas import tpu_sc as plsc

_NC = 2
_NS = 16
_NW = _NC * _NS
_D = 128
_CHUNK = 128


def _sc_gather(x_flat, table, n_rows):
    per_w = n_rows // _NW
    n_chunks = per_w // _CHUNK
    x3 = x_flat.reshape(_NW, n_chunks, _CHUNK)
    mesh = plsc.VectorSubcoreMesh(core_axis_name="c", subcore_axis_name="s")

    @functools.partial(
        pl.kernel,
        out_type=jax.ShapeDtypeStruct((n_rows, _D), jnp.float32),
        mesh=mesh,
        scratch_types=[
            pltpu.VMEM((n_chunks, _CHUNK), jnp.int32),
            pltpu.VMEM((_CHUNK, _D), jnp.float32),
            pltpu.SemaphoreType.DMA,
        ],
    )
    def k(x_hbm, tab_hbm, out_hbm, idx_v, rows_v, sem):
        wid = lax.axis_index("s") * _NC + lax.axis_index("c")
        base = wid * per_w
        pltpu.sync_copy(x_hbm.at[wid], idx_v)

        @pl.loop(0, n_chunks)
        def _(j):
            pltpu.async_copy(tab_hbm.at[idx_v.at[j]], rows_v, sem).wait()
            pltpu.sync_copy(rows_v, out_hbm.at[pl.ds(base + j * _CHUNK, _CHUNK)])

    return k(x3, table)


def kernel(x, embedding_weight):
    b, l = x.shape
    out = _sc_gather(x.reshape(-1), embedding_weight, b * l)
    return out.reshape(b, l, _D)


# speedup vs baseline: 3.3097x; 1.1084x over previous
import functools

import jax
import jax.numpy as jnp
from jax import lax
from jax.experimental import pallas as pl
from jax.experimental.pallas import tpu as pltpu
from jax.experimental.pallas import tpu_sc as plsc

_NC = 2
_NS = 16
_NW = _NC * _NS
_D = 128
_CHUNK = 128
_NBUF = 5


def _sc_gather(x_flat, table, n_rows):
    per_w = n_rows // _NW
    n_chunks = per_w // _CHUNK
    n_groups = n_chunks // _NBUF
    x3 = x_flat.reshape(_NW, n_chunks, _CHUNK)
    mesh = plsc.VectorSubcoreMesh(core_axis_name="c", subcore_axis_name="s")

    @functools.partial(
        pl.kernel,
        out_type=jax.ShapeDtypeStruct((n_rows, _D), jnp.float32),
        mesh=mesh,
        scratch_types=[
            pltpu.VMEM((n_chunks, _CHUNK), jnp.int32),
            [pltpu.VMEM((_CHUNK, _D), jnp.float32) for _ in range(_NBUF)],
            [pltpu.SemaphoreType.DMA for _ in range(_NBUF)],
            [pltpu.SemaphoreType.DMA for _ in range(_NBUF)],
        ],
    )
    def k(x_hbm, tab_hbm, out_hbm, idx_v, bufs, gsems, osems):
        wid = lax.axis_index("s") * _NC + lax.axis_index("c")
        base = wid * per_w
        pltpu.sync_copy(x_hbm.at[wid], idx_v)

        def gather_start(j, b, sem):
            pltpu.async_copy(tab_hbm.at[idx_v.at[j]], bufs[b], sem)

        def gather_wait(j, b, sem):
            pltpu.make_async_copy(tab_hbm.at[idx_v.at[j]], bufs[b], sem).wait()

        def out_ref(j):
            return out_hbm.at[pl.ds(base + j * _CHUNK, _CHUNK)]

        for b in range(_NBUF):
            gather_start(b, b, gsems[b])

        @pl.loop(0, n_groups)
        def _(g):
            j0 = g * _NBUF
            for b in range(_NBUF):
                gather_wait(j0 + b, b, gsems[b])
                pltpu.async_copy(bufs[b], out_ref(j0 + b), osems[b])

            @pl.when(g < n_groups - 1)
            def _():
                for b in range(_NBUF):
                    pltpu.make_async_copy(bufs[b], out_ref(j0 + b), osems[b]).wait()
                    gather_start(j0 + _NBUF + b, b, gsems[b])

        for b in range(_NBUF):
            j = (n_groups - 1) * _NBUF + b
            pltpu.make_async_copy(bufs[b], out_ref(j), osems[b]).wait()

    return k(x3, table)


def kernel(x, embedding_weight):
    b, l = x.shape
    out = _sc_gather(x.reshape(-1), embedding_weight, b * l)
    return out.reshape(b, l, _D)


# speedup vs baseline: 5.9468x; 1.7968x over previous
import functools

import jax
import jax.numpy as jnp
from jax import lax
from jax.experimental import pallas as pl
from jax.experimental.pallas import tpu as pltpu
from jax.experimental.pallas import tpu_sc as plsc

_NC = 2
_NS = 16
_NW = _NC * _NS
_NBUF = 8


def _sc_gather(x, table):
    b, l = x.shape
    d = table.shape[1]
    rows_per_w = b // _NW
    n_groups = rows_per_w // _NBUF
    mesh = plsc.VectorSubcoreMesh(core_axis_name="c", subcore_axis_name="s")

    @functools.partial(
        pl.kernel,
        out_type=jax.ShapeDtypeStruct((b, l, d), jnp.float32),
        mesh=mesh,
        scratch_types=[
            pltpu.VMEM((rows_per_w, l), jnp.int32),
            [pltpu.VMEM((l, d), jnp.float32) for _ in range(_NBUF)],
            [pltpu.SemaphoreType.DMA for _ in range(_NBUF)],
            [pltpu.SemaphoreType.DMA for _ in range(_NBUF)],
        ],
    )
    def k(x_hbm, tab_hbm, out_hbm, idx_v, bufs, gsems, osems):
        wid = lax.axis_index("s") * _NC + lax.axis_index("c")
        base = wid * rows_per_w
        pltpu.sync_copy(x_hbm.at[pl.ds(base, rows_per_w)], idx_v)

        def gather_start(j, bi):
            pltpu.async_copy(tab_hbm.at[idx_v.at[j]], bufs[bi], gsems[bi])

        def gather_wait(j, bi):
            pltpu.make_async_copy(tab_hbm.at[idx_v.at[j]], bufs[bi], gsems[bi]).wait()

        for bi in range(_NBUF):
            gather_start(bi, bi)

        @pl.loop(0, n_groups)
        def _(g):
            j0 = g * _NBUF
            for bi in range(_NBUF):
                gather_wait(j0 + bi, bi)
                pltpu.async_copy(bufs[bi], out_hbm.at[base + j0 + bi], osems[bi])

            @pl.when(g < n_groups - 1)
            def _():
                for bi in range(_NBUF):
                    pltpu.make_async_copy(
                        bufs[bi], out_hbm.at[base + j0 + bi], osems[bi]
                    ).wait()
                    gather_start(j0 + _NBUF + bi, bi)

        for bi in range(_NBUF):
            j = (n_groups - 1) * _NBUF + bi
            pltpu.make_async_copy(bufs[bi], out_hbm.at[base + j], osems[bi]).wait()

    return k(x, table)


def kernel(x, embedding_weight):
    return _sc_gather(x, embedding_weight)
